# baseline (speedup 1.0000x reference)
import functools

import jax
import jax.numpy as jnp
from jax import lax
from jax.experimental import pallas as pl
from jax.experimental.pallas import tpu as pltpu
from jax.experimental.pallas import tpu_sc as plsc

NG = 512
NA = 20000
NC = 2
NS = 16
NW = NC * NS
APW = 640
NAP = NW * APW
LASTW = NA - (NW - 1) * APW
NT = NG // 16
NCH = APW // 16
BG = 0.4
FG = 0.5

_MESH = plsc.VectorSubcoreMesh(core_axis_name="c", subcore_axis_name="s")
_PARAMS = pltpu.CompilerParams(use_tc_tiling_on_sc=False,
                               needs_layout_passes=False)


def _bcast(v, j):
    idx = jnp.full((16,), j, jnp.int32)
    return v.at[idx].get(mode="promise_in_bounds")


def _iou16(a1, a2, a3, a4, aar, g1b, g2b, g3b, g4b, gab):
    ltx = jnp.maximum(a1, g1b)
    lty = jnp.maximum(a2, g2b)
    rbx = jnp.minimum(a3, g3b)
    rby = jnp.minimum(a4, g4b)
    w = jnp.maximum(rbx - ltx, 0.0)
    h = jnp.maximum(rby - lty, 0.0)
    inter = w * h
    union = (gab + aar) - inter
    return inter / union


@functools.partial(
    pl.kernel,
    out_type=[
        jax.ShapeDtypeStruct((NG, NA), jnp.float32),
        jax.ShapeDtypeStruct((NAP,), jnp.float32),
        jax.ShapeDtypeStruct((NAP,), jnp.int32),
        jax.ShapeDtypeStruct((NW * NG,), jnp.float32),
    ],
    mesh=_MESH,
    compiler_params=_PARAMS,
    scratch_types=[
        pltpu.VMEM((NG,), jnp.float32),
        pltpu.VMEM((NG,), jnp.float32),
        pltpu.VMEM((NG,), jnp.float32),
        pltpu.VMEM((NG,), jnp.float32),
        pltpu.VMEM((NG,), jnp.float32),
        pltpu.VMEM((APW,), jnp.float32),
        pltpu.VMEM((APW,), jnp.float32),
        pltpu.VMEM((APW,), jnp.float32),
        pltpu.VMEM((APW,), jnp.float32),
        pltpu.VMEM((APW,), jnp.float32),
        pltpu.VMEM((APW,), jnp.float32),
        pltpu.VMEM((APW,), jnp.int32),
        pltpu.VMEM((16, APW), jnp.float32),
        pltpu.VMEM((NG,), jnp.float32),
    ],
)
def _pass_a(gx1_h, gy1_h, gx2_h, gy2_h, ax1_h, ay1_h, ax2_h, ay2_h,
            mat_h, cmax_h, cidx_h, rmax_h,
            gx1, gy1, gx2, gy2, ga, ax1, ay1, ax2, ay2, aa,
            cmv, civ, buf, rmw):
    wid = lax.axis_index("s") * NC + lax.axis_index("c")
    base = wid * APW

    pltpu.sync_copy(gx1_h, gx1)
    pltpu.sync_copy(gy1_h, gy1)
    pltpu.sync_copy(gx2_h, gx2)
    pltpu.sync_copy(gy2_h, gy2)
    pltpu.sync_copy(ax1_h.at[pl.ds(base, APW)], ax1)
    pltpu.sync_copy(ay1_h.at[pl.ds(base, APW)], ay1)
    pltpu.sync_copy(ax2_h.at[pl.ds(base, APW)], ax2)
    pltpu.sync_copy(ay2_h.at[pl.ds(base, APW)], ay2)

    def _garea(t, _):
        s = pl.ds(t * 16, 16)
        ga[s] = (gx2[s] - gx1[s]) * (gy2[s] - gy1[s])
        return 0
    lax.fori_loop(0, NT, _garea, 0)

    def _aarea(c, _):
        s = pl.ds(c * 16, 16)
        aa[s] = (ax2[s] - ax1[s]) * (ay2[s] - ay1[s])
        cmv[s] = jnp.full((16,), -1.0, jnp.float32)
        civ[s] = jnp.zeros((16,), jnp.int32)
        return 0
    lax.fori_loop(0, NCH, _aarea, 0)

    lane = jnp.arange(16, dtype=jnp.int32)
    zeros16 = jnp.zeros((16,), jnp.float32)

    def _gt_group(t, _):
        gs = pl.ds(t * 16, 16)
        g1 = gx1[gs]
        g2 = gy1[gs]
        g3 = gx2[gs]
        g4 = gy2[gs]
        gar = ga[gs]

        racc = [None] * 16
        for j in range(16):
            g1b = _bcast(g1, j)
            g2b = _bcast(g2, j)
            g3b = _bcast(g3, j)
            g4b = _bcast(g4, j)
            gab = _bcast(gar, j)
            gidx = t * 16 + j

            def _chunk(c, rj, j=j, g1b=g1b, g2b=g2b, g3b=g3b, g4b=g4b,
                       gab=gab, gidx=gidx):
                s = pl.ds(c * 16, 16)
                iou = _iou16(ax1[s], ay1[s], ax2[s], ay2[s], aa[s],
                             g1b, g2b, g3b, g4b, gab)
                buf[j, s] = iou
                cm = cmv[s]
                civ[s] = jnp.where(iou > cm, gidx, civ[s])
                cmv[s] = jnp.maximum(cm, iou)
                return jnp.maximum(rj, iou)

            racc[j] = plsc.parallel_loop(0, NCH, unroll=4,
                                         carry=zeros16)(_chunk)

        @pl.when(wid != NW - 1)
        def _():
            pltpu.sync_copy(buf, mat_h.at[pl.ds(t * 16, 16), pl.ds(base, APW)])

        @pl.when(wid == NW - 1)
        def _():
            pltpu.sync_copy(buf.at[:, pl.ds(0, LASTW)],
                            mat_h.at[pl.ds(t * 16, 16), pl.ds(NA - LASTW, LASTW)])

        acc = zeros16
        for j in range(16):
            m = jnp.max(racc[j])
            acc = jnp.where(lane == j, m, acc)
        rmw[gs] = acc
        return 0

    lax.fori_loop(0, NT, _gt_group, 0)

    pltpu.sync_copy(cmv, cmax_h.at[pl.ds(base, APW)])
    pltpu.sync_copy(civ, cidx_h.at[pl.ds(base, APW)])
    pltpu.sync_copy(rmw, rmax_h.at[pl.ds(wid * NG, NG)])


@functools.partial(
    pl.kernel,
    out_type=[jax.ShapeDtypeStruct((NAP,), jnp.int32)],
    mesh=_MESH,
    compiler_params=_PARAMS,
    scratch_types=[
        pltpu.VMEM((NG,), jnp.float32),
        pltpu.VMEM((NG,), jnp.float32),
        pltpu.VMEM((NG,), jnp.float32),
        pltpu.VMEM((NG,), jnp.float32),
        pltpu.VMEM((NG,), jnp.float32),
        pltpu.VMEM((APW,), jnp.float32),
        pltpu.VMEM((APW,), jnp.float32),
        pltpu.VMEM((APW,), jnp.float32),
        pltpu.VMEM((APW,), jnp.float32),
        pltpu.VMEM((APW,), jnp.float32),
        pltpu.VMEM((NW * NG,), jnp.float32),
        pltpu.VMEM((NG,), jnp.float32),
        pltpu.VMEM((APW,), jnp.float32),
        pltpu.VMEM((APW,), jnp.int32),
        pltpu.VMEM((APW,), jnp.int32),
        pltpu.VMEM((APW,), jnp.int32),
    ],
)
def _pass_b(gx1_h, gy1_h, gx2_h, gy2_h, ax1_h, ay1_h, ax2_h, ay2_h,
            rmax_h, cmax_h, cidx_h, idx_h,
            gx1, gy1, gx2, gy2, ga, ax1, ay1, ax2, ay2, aa,
            rall, grm, cmv, civ, resv, outv):
    wid = lax.axis_index("s") * NC + lax.axis_index("c")
    base = wid * APW

    pltpu.sync_copy(gx1_h, gx1)
    pltpu.sync_copy(gy1_h, gy1)
    pltpu.sync_copy(gx2_h, gx2)
    pltpu.sync_copy(gy2_h, gy2)
    pltpu.sync_copy(ax1_h.at[pl.ds(base, APW)], ax1)
    pltpu.sync_copy(ay1_h.at[pl.ds(base, APW)], ay1)
    pltpu.sync_copy(ax2_h.at[pl.ds(base, APW)], ax2)
    pltpu.sync_copy(ay2_h.at[pl.ds(base, APW)], ay2)
    pltpu.sync_copy(rmax_h, rall)
    pltpu.sync_copy(cmax_h.at[pl.ds(base, APW)], cmv)
    pltpu.sync_copy(cidx_h.at[pl.ds(base, APW)], civ)

    def _garea(t, _):
        s = pl.ds(t * 16, 16)
        ga[s] = (gx2[s] - gx1[s]) * (gy2[s] - gy1[s])
        return 0
    lax.fori_loop(0, NT, _garea, 0)

    def _aareab(c, _):
        s = pl.ds(c * 16, 16)
        aa[s] = (ax2[s] - ax1[s]) * (ay2[s] - ay1[s])
        resv[s] = jnp.zeros((16,), jnp.int32)
        return 0
    lax.fori_loop(0, NCH, _aareab, 0)

    def _reduce(t, _):
        acc = rall[pl.ds(t * 16, 16)]
        for w in range(1, NW):
            acc = jnp.maximum(acc, rall[pl.ds(w * NG + t * 16, 16)])
        grm[pl.ds(t * 16, 16)] = acc
        return 0
    lax.fori_loop(0, NT, _reduce, 0)

    lane = jnp.arange(16, dtype=jnp.int32)
    ones16 = jnp.ones((16,), jnp.int32)

    def _mark(t, _):
        gs = pl.ds(t * 16, 16)
        grmv = grm[gs]
        ownv = rall[pl.ds(wid * NG + t * 16, 16)]
        eq = ownv == grmv
        g1 = gx1[gs]
        g2 = gy1[gs]
        g3 = gx2[gs]
        g4 = gy2[gs]
        gar = ga[gs]
        for j in range(16):
            cand = jnp.any(eq & (lane == j))

            @pl.when(cand)
            def _(j=j, g1=g1, g2=g2, g3=g3, g4=g4, gar=gar, grmv=grmv):
                g1b = _bcast(g1, j)
                g2b = _bcast(g2, j)
                g3b = _bcast(g3, j)
                g4b = _bcast(g4, j)
                gab = _bcast(gar, j)
                gmb = _bcast(grmv, j)

                def _scan(c, _):
                    s = pl.ds(c * 16, 16)
                    iou = _iou16(ax1[s], ay1[s], ax2[s], ay2[s], aa[s],
                                 g1b, g2b, g3b, g4b, gab)
                    plsc.store_scatter(resv, [lane + c * 16], ones16,
                                       mask=iou == gmb)
                    return 0
                lax.fori_loop(0, NCH, _scan, 0)
        return 0

    lax.fori_loop(0, NT, _mark, 0)

    def _final(c, _):
        s = pl.ds(c * 16, 16)
        mx = cmv[s]
        ix = civ[s]
        m = jnp.where(mx < BG, jnp.int32(-1), ix)
        m = jnp.where((mx >= BG) & (mx < FG), jnp.int32(-2), m)
        m = jnp.where(resv[s] > 0, ix, m)
        outv[s] = m
        return 0
    lax.fori_loop(0, NCH, _final, 0)

    pltpu.sync_copy(outv, idx_h.at[pl.ds(base, APW)])


def kernel(gt_boxes, anchors):
    ap = jnp.concatenate(
        [anchors, jnp.zeros((NAP - NA, 4), jnp.float32)], axis=0)
    gcols = (gt_boxes[:, 0], gt_boxes[:, 1], gt_boxes[:, 2], gt_boxes[:, 3])
    acols = (ap[:, 0], ap[:, 1], ap[:, 2], ap[:, 3])
    mat, cmax, cidx, rmax = _pass_a(*gcols, *acols)
    idxp, = _pass_b(*gcols, *acols, rmax, cmax, cidx)
    return mat, idxp[:NA].astype(jnp.int64)


# speedup vs baseline: 1.1426x; 1.1426x over previous
import functools

import jax
import jax.numpy as jnp
from jax import lax
from jax.experimental import pallas as pl
from jax.experimental.pallas import tpu as pltpu
from jax.experimental.pallas import tpu_sc as plsc

NG = 512
NA = 20000
NC = 2
NS = 16
NW = NC * NS
APW = 640
NAP = NW * APW
LASTW = NA - (NW - 1) * APW
NT = NG // 16
NCH = APW // 16
BG = 0.4
FG = 0.5

_MESH = plsc.VectorSubcoreMesh(core_axis_name="c", subcore_axis_name="s")
_PARAMS = pltpu.CompilerParams(use_tc_tiling_on_sc=False,
                               needs_layout_passes=False)


def _bcast(v, j):
    idx = jnp.full((16,), j, jnp.int32)
    return v.at[idx].get(mode="promise_in_bounds")


def _iou16(a1, a2, a3, a4, aar, g1b, g2b, g3b, g4b, gab):
    ltx = jnp.maximum(a1, g1b)
    lty = jnp.maximum(a2, g2b)
    rbx = jnp.minimum(a3, g3b)
    rby = jnp.minimum(a4, g4b)
    w = jnp.maximum(rbx - ltx, 0.0)
    h = jnp.maximum(rby - lty, 0.0)
    inter = w * h
    union = (gab + aar) - inter
    return inter / union


@functools.partial(
    pl.kernel,
    out_type=[
        jax.ShapeDtypeStruct((NG, NA), jnp.float32),
        jax.ShapeDtypeStruct((NAP,), jnp.float32),
        jax.ShapeDtypeStruct((NAP,), jnp.int32),
        jax.ShapeDtypeStruct((NW * NG,), jnp.float32),
    ],
    mesh=_MESH,
    compiler_params=_PARAMS,
    scratch_types=[
        pltpu.VMEM((NG,), jnp.float32),
        pltpu.VMEM((NG,), jnp.float32),
        pltpu.VMEM((NG,), jnp.float32),
        pltpu.VMEM((NG,), jnp.float32),
        pltpu.VMEM((NG,), jnp.float32),
        pltpu.VMEM((APW,), jnp.float32),
        pltpu.VMEM((APW,), jnp.float32),
        pltpu.VMEM((APW,), jnp.float32),
        pltpu.VMEM((APW,), jnp.float32),
        pltpu.VMEM((APW,), jnp.float32),
        pltpu.VMEM((APW,), jnp.float32),
        pltpu.VMEM((APW,), jnp.int32),
        pltpu.VMEM((16, APW), jnp.float32),
        pltpu.VMEM((NG,), jnp.float32),
    ],
)
def _pass_a(gx1_h, gy1_h, gx2_h, gy2_h, ax1_h, ay1_h, ax2_h, ay2_h,
            mat_h, cmax_h, cidx_h, rmax_h,
            gx1, gy1, gx2, gy2, ga, ax1, ay1, ax2, ay2, aa,
            cmv, civ, buf, rmw):
    wid = lax.axis_index("s") * NC + lax.axis_index("c")
    base = wid * APW

    pltpu.sync_copy(gx1_h, gx1)
    pltpu.sync_copy(gy1_h, gy1)
    pltpu.sync_copy(gx2_h, gx2)
    pltpu.sync_copy(gy2_h, gy2)
    pltpu.sync_copy(ax1_h.at[pl.ds(base, APW)], ax1)
    pltpu.sync_copy(ay1_h.at[pl.ds(base, APW)], ay1)
    pltpu.sync_copy(ax2_h.at[pl.ds(base, APW)], ax2)
    pltpu.sync_copy(ay2_h.at[pl.ds(base, APW)], ay2)

    def _garea(t, _):
        s = pl.ds(t * 16, 16)
        ga[s] = (gx2[s] - gx1[s]) * (gy2[s] - gy1[s])
        return 0
    lax.fori_loop(0, NT, _garea, 0)

    def _aarea(c, _):
        s = pl.ds(c * 16, 16)
        aa[s] = (ax2[s] - ax1[s]) * (ay2[s] - ay1[s])
        cmv[s] = jnp.full((16,), -1.0, jnp.float32)
        civ[s] = jnp.zeros((16,), jnp.int32)
        return 0
    lax.fori_loop(0, NCH, _aarea, 0)

    lane = jnp.arange(16, dtype=jnp.int32)
    zeros16 = jnp.zeros((16,), jnp.float32)

    def _gt_group(t, _):
        gs = pl.ds(t * 16, 16)
        g1 = gx1[gs]
        g2 = gy1[gs]
        g3 = gx2[gs]
        g4 = gy2[gs]
        gar = ga[gs]

        racc = [None] * 16
        for j in range(16):
            g1b = _bcast(g1, j)
            g2b = _bcast(g2, j)
            g3b = _bcast(g3, j)
            g4b = _bcast(g4, j)
            gab = _bcast(gar, j)
            gidx = t * 16 + j

            def _chunk(c, rj, j=j, g1b=g1b, g2b=g2b, g3b=g3b, g4b=g4b,
                       gab=gab, gidx=gidx):
                s = pl.ds(c * 16, 16)
                iou = _iou16(ax1[s], ay1[s], ax2[s], ay2[s], aa[s],
                             g1b, g2b, g3b, g4b, gab)
                buf[j, s] = iou
                cm = cmv[s]
                civ[s] = jnp.where(iou > cm, gidx, civ[s])
                cmv[s] = jnp.maximum(cm, iou)
                return jnp.maximum(rj, iou)

            racc[j] = plsc.parallel_loop(0, NCH, unroll=2,
                                         carry=zeros16)(_chunk)

        @pl.when(wid != NW - 1)
        def _():
            pltpu.sync_copy(buf, mat_h.at[pl.ds(t * 16, 16), pl.ds(base, APW)])

        @pl.when(wid == NW - 1)
        def _():
            pltpu.sync_copy(buf.at[:, pl.ds(0, LASTW)],
                            mat_h.at[pl.ds(t * 16, 16), pl.ds(NA - LASTW, LASTW)])

        acc = zeros16
        for j in range(16):
            m = jnp.max(racc[j])
            acc = jnp.where(lane == j, m, acc)
        rmw[gs] = acc
        return 0

    lax.fori_loop(0, NT, _gt_group, 0)

    pltpu.sync_copy(cmv, cmax_h.at[pl.ds(base, APW)])
    pltpu.sync_copy(civ, cidx_h.at[pl.ds(base, APW)])
    pltpu.sync_copy(rmw, rmax_h.at[pl.ds(wid * NG, NG)])


@functools.partial(
    pl.kernel,
    out_type=[jax.ShapeDtypeStruct((NAP,), jnp.int32)],
    mesh=_MESH,
    compiler_params=_PARAMS,
    scratch_types=[
        pltpu.VMEM((NG,), jnp.float32),
        pltpu.VMEM((NG,), jnp.float32),
        pltpu.VMEM((NG,), jnp.float32),
        pltpu.VMEM((NG,), jnp.float32),
        pltpu.VMEM((NG,), jnp.float32),
        pltpu.VMEM((APW,), jnp.float32),
        pltpu.VMEM((APW,), jnp.float32),
        pltpu.VMEM((APW,), jnp.float32),
        pltpu.VMEM((APW,), jnp.float32),
        pltpu.VMEM((APW,), jnp.float32),
        pltpu.VMEM((NW * NG,), jnp.float32),
        pltpu.VMEM((NG,), jnp.float32),
        pltpu.VMEM((APW,), jnp.float32),
        pltpu.VMEM((APW,), jnp.int32),
        pltpu.VMEM((APW,), jnp.int32),
        pltpu.VMEM((APW,), jnp.int32),
    ],
)
def _pass_b(gx1_h, gy1_h, gx2_h, gy2_h, ax1_h, ay1_h, ax2_h, ay2_h,
            rmax_h, cmax_h, cidx_h, idx_h,
            gx1, gy1, gx2, gy2, ga, ax1, ay1, ax2, ay2, aa,
            rall, grm, cmv, civ, resv, outv):
    wid = lax.axis_index("s") * NC + lax.axis_index("c")
    base = wid * APW

    pltpu.sync_copy(gx1_h, gx1)
    pltpu.sync_copy(gy1_h, gy1)
    pltpu.sync_copy(gx2_h, gx2)
    pltpu.sync_copy(gy2_h, gy2)
    pltpu.sync_copy(ax1_h.at[pl.ds(base, APW)], ax1)
    pltpu.sync_copy(ay1_h.at[pl.ds(base, APW)], ay1)
    pltpu.sync_copy(ax2_h.at[pl.ds(base, APW)], ax2)
    pltpu.sync_copy(ay2_h.at[pl.ds(base, APW)], ay2)
    pltpu.sync_copy(rmax_h, rall)
    pltpu.sync_copy(cmax_h.at[pl.ds(base, APW)], cmv)
    pltpu.sync_copy(cidx_h.at[pl.ds(base, APW)], civ)

    def _garea(t, _):
        s = pl.ds(t * 16, 16)
        ga[s] = (gx2[s] - gx1[s]) * (gy2[s] - gy1[s])
        return 0
    lax.fori_loop(0, NT, _garea, 0)

    def _aareab(c, _):
        s = pl.ds(c * 16, 16)
        aa[s] = (ax2[s] - ax1[s]) * (ay2[s] - ay1[s])
        resv[s] = jnp.zeros((16,), jnp.int32)
        return 0
    lax.fori_loop(0, NCH, _aareab, 0)

    def _reduce(t, _):
        acc = rall[pl.ds(t * 16, 16)]
        for w in range(1, NW):
            acc = jnp.maximum(acc, rall[pl.ds(w * NG + t * 16, 16)])
        grm[pl.ds(t * 16, 16)] = acc
        return 0
    lax.fori_loop(0, NT, _reduce, 0)

    lane = jnp.arange(16, dtype=jnp.int32)
    ones16 = jnp.ones((16,), jnp.int32)

    def _mark(t, _):
        gs = pl.ds(t * 16, 16)
        grmv = grm[gs]
        ownv = rall[pl.ds(wid * NG + t * 16, 16)]
        eq = ownv == grmv
        g1 = gx1[gs]
        g2 = gy1[gs]
        g3 = gx2[gs]
        g4 = gy2[gs]
        gar = ga[gs]
        for j in range(16):
            cand = jnp.any(eq & (lane == j))

            @pl.when(cand)
            def _(j=j, g1=g1, g2=g2, g3=g3, g4=g4, gar=gar, grmv=grmv):
                g1b = _bcast(g1, j)
                g2b = _bcast(g2, j)
                g3b = _bcast(g3, j)
                g4b = _bcast(g4, j)
                gab = _bcast(gar, j)
                gmb = _bcast(grmv, j)

                def _scan(c, _):
                    s = pl.ds(c * 16, 16)
                    iou = _iou16(ax1[s], ay1[s], ax2[s], ay2[s], aa[s],
                                 g1b, g2b, g3b, g4b, gab)
                    plsc.store_scatter(resv, [lane + c * 16], ones16,
                                       mask=iou == gmb)
                    return 0
                lax.fori_loop(0, NCH, _scan, 0)
        return 0

    lax.fori_loop(0, NT, _mark, 0)

    def _final(c, _):
        s = pl.ds(c * 16, 16)
        mx = cmv[s]
        ix = civ[s]
        m = jnp.where(mx < BG, jnp.int32(-1), ix)
        m = jnp.where((mx >= BG) & (mx < FG), jnp.int32(-2), m)
        m = jnp.where(resv[s] > 0, ix, m)
        outv[s] = m
        return 0
    lax.fori_loop(0, NCH, _final, 0)

    pltpu.sync_copy(outv, idx_h.at[pl.ds(base, APW)])


def kernel(gt_boxes, anchors):
    ap = jnp.concatenate(
        [anchors, jnp.zeros((NAP - NA, 4), jnp.float32)], axis=0)
    gcols = (gt_boxes[:, 0], gt_boxes[:, 1], gt_boxes[:, 2], gt_boxes[:, 3])
    acols = (ap[:, 0], ap[:, 1], ap[:, 2], ap[:, 3])
    mat, cmax, cidx, rmax = _pass_a(*gcols, *acols)
    idxp, = _pass_b(*gcols, *acols, rmax, cmax, cidx)
    return mat, idxp[:NA].astype(jnp.int64)


# speedup vs baseline: 1.1885x; 1.0402x over previous
import functools

import jax
import jax.numpy as jnp
from jax import lax
from jax.experimental import pallas as pl
from jax.experimental.pallas import tpu as pltpu
from jax.experimental.pallas import tpu_sc as plsc

NG = 512
NA = 20000
NC = 2
NS = 16
NW = NC * NS
APW = 640
NAP = NW * APW
LASTW = NA - (NW - 1) * APW
NT = NG // 16
NCH = APW // 16
BG = 0.4
FG = 0.5

_MESH = plsc.VectorSubcoreMesh(core_axis_name="c", subcore_axis_name="s")
_PARAMS = pltpu.CompilerParams(use_tc_tiling_on_sc=False,
                               needs_layout_passes=False)


def _bcast(v, j):
    idx = jnp.full((16,), j, jnp.int32)
    return v.at[idx].get(mode="promise_in_bounds")


def _iou16(a1, a2, a3, a4, aar, g1b, g2b, g3b, g4b, gab):
    ltx = jnp.maximum(a1, g1b)
    lty = jnp.maximum(a2, g2b)
    rbx = jnp.minimum(a3, g3b)
    rby = jnp.minimum(a4, g4b)
    w = jnp.maximum(rbx - ltx, 0.0)
    h = jnp.maximum(rby - lty, 0.0)
    inter = w * h
    union = (gab + aar) - inter
    return inter / union


@functools.partial(
    pl.kernel,
    out_type=[
        jax.ShapeDtypeStruct((NG, NA), jnp.float32),
        jax.ShapeDtypeStruct((NAP,), jnp.float32),
        jax.ShapeDtypeStruct((NAP,), jnp.int32),
        jax.ShapeDtypeStruct((NW * NG,), jnp.float32),
    ],
    mesh=_MESH,
    compiler_params=_PARAMS,
    scratch_types=[
        pltpu.VMEM((NG,), jnp.float32),
        pltpu.VMEM((NG,), jnp.float32),
        pltpu.VMEM((NG,), jnp.float32),
        pltpu.VMEM((NG,), jnp.float32),
        pltpu.VMEM((NG,), jnp.float32),
        pltpu.VMEM((APW,), jnp.float32),
        pltpu.VMEM((APW,), jnp.float32),
        pltpu.VMEM((APW,), jnp.float32),
        pltpu.VMEM((APW,), jnp.float32),
        pltpu.VMEM((APW,), jnp.float32),
        pltpu.VMEM((APW,), jnp.float32),
        pltpu.VMEM((APW,), jnp.int32),
        pltpu.VMEM((16, APW), jnp.float32),
        pltpu.VMEM((NG,), jnp.float32),
    ],
)
def _pass_a(gx1_h, gy1_h, gx2_h, gy2_h, ax1_h, ay1_h, ax2_h, ay2_h,
            mat_h, cmax_h, cidx_h, rmax_h,
            gx1, gy1, gx2, gy2, ga, ax1, ay1, ax2, ay2, aa,
            cmv, civ, buf, rmw):
    wid = lax.axis_index("s") * NC + lax.axis_index("c")
    base = wid * APW

    pltpu.sync_copy(gx1_h, gx1)
    pltpu.sync_copy(gy1_h, gy1)
    pltpu.sync_copy(gx2_h, gx2)
    pltpu.sync_copy(gy2_h, gy2)
    pltpu.sync_copy(ax1_h.at[pl.ds(base, APW)], ax1)
    pltpu.sync_copy(ay1_h.at[pl.ds(base, APW)], ay1)
    pltpu.sync_copy(ax2_h.at[pl.ds(base, APW)], ax2)
    pltpu.sync_copy(ay2_h.at[pl.ds(base, APW)], ay2)

    def _garea(t, _):
        s = pl.ds(t * 16, 16)
        ga[s] = (gx2[s] - gx1[s]) * (gy2[s] - gy1[s])
        return 0
    lax.fori_loop(0, NT, _garea, 0)

    def _aarea(c, _):
        s = pl.ds(c * 16, 16)
        aa[s] = (ax2[s] - ax1[s]) * (ay2[s] - ay1[s])
        cmv[s] = jnp.full((16,), -1.0, jnp.float32)
        civ[s] = jnp.zeros((16,), jnp.int32)
        return 0
    lax.fori_loop(0, NCH, _aarea, 0)

    lane = jnp.arange(16, dtype=jnp.int32)
    zeros16 = jnp.zeros((16,), jnp.float32)

    def _gt_group(t, _):
        gs = pl.ds(t * 16, 16)
        g1 = gx1[gs]
        g2 = gy1[gs]
        g3 = gx2[gs]
        g4 = gy2[gs]
        gar = ga[gs]

        racc = [None] * 16
        for j in range(16):
            g1b = _bcast(g1, j)
            g2b = _bcast(g2, j)
            g3b = _bcast(g3, j)
            g4b = _bcast(g4, j)
            gab = _bcast(gar, j)
            gidx = t * 16 + j

            def _chunk(c, rj, j=j, g1b=g1b, g2b=g2b, g3b=g3b, g4b=g4b,
                       gab=gab):
                s = pl.ds(c * 16, 16)
                iou = _iou16(ax1[s], ay1[s], ax2[s], ay2[s], aa[s],
                             g1b, g2b, g3b, g4b, gab)
                buf[j, s] = iou
                return jnp.maximum(rj, iou)

            racc[j] = plsc.parallel_loop(0, NCH, unroll=2,
                                         carry=zeros16)(_chunk)

        def _cmx(c, t=t):
            s = pl.ds(c * 16, 16)
            cm = cmv[s]
            ci = civ[s]
            for j in range(16):
                v = buf[j, s]
                ci = jnp.where(v > cm, t * 16 + j, ci)
                cm = jnp.maximum(cm, v)
            cmv[s] = cm
            civ[s] = ci
        plsc.parallel_loop(0, NCH, unroll=2)(_cmx)

        @pl.when(wid != NW - 1)
        def _():
            pltpu.sync_copy(buf, mat_h.at[pl.ds(t * 16, 16), pl.ds(base, APW)])

        @pl.when(wid == NW - 1)
        def _():
            pltpu.sync_copy(buf.at[:, pl.ds(0, LASTW)],
                            mat_h.at[pl.ds(t * 16, 16), pl.ds(NA - LASTW, LASTW)])

        acc = zeros16
        for j in range(16):
            m = jnp.max(racc[j])
            acc = jnp.where(lane == j, m, acc)
        rmw[gs] = acc
        return 0

    lax.fori_loop(0, NT, _gt_group, 0)

    pltpu.sync_copy(cmv, cmax_h.at[pl.ds(base, APW)])
    pltpu.sync_copy(civ, cidx_h.at[pl.ds(base, APW)])
    pltpu.sync_copy(rmw, rmax_h.at[pl.ds(wid * NG, NG)])


@functools.partial(
    pl.kernel,
    out_type=[jax.ShapeDtypeStruct((NAP,), jnp.int32)],
    mesh=_MESH,
    compiler_params=_PARAMS,
    scratch_types=[
        pltpu.VMEM((NG,), jnp.float32),
        pltpu.VMEM((NG,), jnp.float32),
        pltpu.VMEM((NG,), jnp.float32),
        pltpu.VMEM((NG,), jnp.float32),
        pltpu.VMEM((NG,), jnp.float32),
        pltpu.VMEM((APW,), jnp.float32),
        pltpu.VMEM((APW,), jnp.float32),
        pltpu.VMEM((APW,), jnp.float32),
        pltpu.VMEM((APW,), jnp.float32),
        pltpu.VMEM((APW,), jnp.float32),
        pltpu.VMEM((NW * NG,), jnp.float32),
        pltpu.VMEM((NG,), jnp.float32),
        pltpu.VMEM((APW,), jnp.float32),
        pltpu.VMEM((APW,), jnp.int32),
        pltpu.VMEM((APW,), jnp.int32),
        pltpu.VMEM((APW,), jnp.int32),
    ],
)
def _pass_b(gx1_h, gy1_h, gx2_h, gy2_h, ax1_h, ay1_h, ax2_h, ay2_h,
            rmax_h, cmax_h, cidx_h, idx_h,
            gx1, gy1, gx2, gy2, ga, ax1, ay1, ax2, ay2, aa,
            rall, grm, cmv, civ, resv, outv):
    wid = lax.axis_index("s") * NC + lax.axis_index("c")
    base = wid * APW

    pltpu.sync_copy(gx1_h, gx1)
    pltpu.sync_copy(gy1_h, gy1)
    pltpu.sync_copy(gx2_h, gx2)
    pltpu.sync_copy(gy2_h, gy2)
    pltpu.sync_copy(ax1_h.at[pl.ds(base, APW)], ax1)
    pltpu.sync_copy(ay1_h.at[pl.ds(base, APW)], ay1)
    pltpu.sync_copy(ax2_h.at[pl.ds(base, APW)], ax2)
    pltpu.sync_copy(ay2_h.at[pl.ds(base, APW)], ay2)
    pltpu.sync_copy(rmax_h, rall)
    pltpu.sync_copy(cmax_h.at[pl.ds(base, APW)], cmv)
    pltpu.sync_copy(cidx_h.at[pl.ds(base, APW)], civ)

    def _garea(t, _):
        s = pl.ds(t * 16, 16)
        ga[s] = (gx2[s] - gx1[s]) * (gy2[s] - gy1[s])
        return 0
    lax.fori_loop(0, NT, _garea, 0)

    def _aareab(c, _):
        s = pl.ds(c * 16, 16)
        aa[s] = (ax2[s] - ax1[s]) * (ay2[s] - ay1[s])
        resv[s] = jnp.zeros((16,), jnp.int32)
        return 0
    lax.fori_loop(0, NCH, _aareab, 0)

    def _reduce(t, _):
        acc = rall[pl.ds(t * 16, 16)]
        for w in range(1, NW):
            acc = jnp.maximum(acc, rall[pl.ds(w * NG + t * 16, 16)])
        grm[pl.ds(t * 16, 16)] = acc
        return 0
    lax.fori_loop(0, NT, _reduce, 0)

    lane = jnp.arange(16, dtype=jnp.int32)
    ones16 = jnp.ones((16,), jnp.int32)

    def _mark(t, _):
        gs = pl.ds(t * 16, 16)
        grmv = grm[gs]
        ownv = rall[pl.ds(wid * NG + t * 16, 16)]
        eq = ownv == grmv
        g1 = gx1[gs]
        g2 = gy1[gs]
        g3 = gx2[gs]
        g4 = gy2[gs]
        gar = ga[gs]
        for j in range(16):
            cand = jnp.any(eq & (lane == j))

            @pl.when(cand)
            def _(j=j, g1=g1, g2=g2, g3=g3, g4=g4, gar=gar, grmv=grmv):
                g1b = _bcast(g1, j)
                g2b = _bcast(g2, j)
                g3b = _bcast(g3, j)
                g4b = _bcast(g4, j)
                gab = _bcast(gar, j)
                gmb = _bcast(grmv, j)

                def _scan(c, _):
                    s = pl.ds(c * 16, 16)
                    iou = _iou16(ax1[s], ay1[s], ax2[s], ay2[s], aa[s],
                                 g1b, g2b, g3b, g4b, gab)
                    plsc.store_scatter(resv, [lane + c * 16], ones16,
                                       mask=iou == gmb)
                    return 0
                lax.fori_loop(0, NCH, _scan, 0)
        return 0

    lax.fori_loop(0, NT, _mark, 0)

    def _final(c, _):
        s = pl.ds(c * 16, 16)
        mx = cmv[s]
        ix = civ[s]
        m = jnp.where(mx < BG, jnp.int32(-1), ix)
        m = jnp.where((mx >= BG) & (mx < FG), jnp.int32(-2), m)
        m = jnp.where(resv[s] > 0, ix, m)
        outv[s] = m
        return 0
    lax.fori_loop(0, NCH, _final, 0)

    pltpu.sync_copy(outv, idx_h.at[pl.ds(base, APW)])


def kernel(gt_boxes, anchors):
    ap = jnp.concatenate(
        [anchors, jnp.zeros((NAP - NA, 4), jnp.float32)], axis=0)
    gcols = (gt_boxes[:, 0], gt_boxes[:, 1], gt_boxes[:, 2], gt_boxes[:, 3])
    acols = (ap[:, 0], ap[:, 1], ap[:, 2], ap[:, 3])
    mat, cmax, cidx, rmax = _pass_a(*gcols, *acols)
    idxp, = _pass_b(*gcols, *acols, rmax, cmax, cidx)
    return mat, idxp[:NA].astype(jnp.int64)


# speedup vs baseline: 1.3763x; 1.1580x over previous
import functools

import jax
import jax.numpy as jnp
from jax import lax
from jax.experimental import pallas as pl
from jax.experimental.pallas import tpu as pltpu
from jax.experimental.pallas import tpu_sc as plsc

NG = 512
NA = 20000
NC = 2
NS = 16
NW = NC * NS
APW = 640
NAP = NW * APW
LASTW = NA - (NW - 1) * APW
NT = NG // 16
NCH = APW // 16
BG = 0.4
FG = 0.5

_MESH = plsc.VectorSubcoreMesh(core_axis_name="c", subcore_axis_name="s")
_PARAMS = pltpu.CompilerParams(needs_layout_passes=False)


def _bcast(v, j):
    idx = jnp.full((16,), j, jnp.int32)
    return v.at[idx].get(mode="promise_in_bounds")


def _iou16(a1, a2, a3, a4, aar, g1b, g2b, g3b, g4b, gab):
    ltx = jnp.maximum(a1, g1b)
    lty = jnp.maximum(a2, g2b)
    rbx = jnp.minimum(a3, g3b)
    rby = jnp.minimum(a4, g4b)
    w = jnp.maximum(rbx - ltx, 0.0)
    h = jnp.maximum(rby - lty, 0.0)
    inter = w * h
    union = (gab + aar) - inter
    return inter / union


@functools.partial(
    pl.kernel,
    out_type=[
        jax.ShapeDtypeStruct((NG, NA), jnp.float32),
        jax.ShapeDtypeStruct((NAP,), jnp.float32),
        jax.ShapeDtypeStruct((NAP,), jnp.int32),
        jax.ShapeDtypeStruct((NW * NG,), jnp.float32),
        jax.ShapeDtypeStruct((NG * 32,), jnp.float32),
    ],
    mesh=_MESH,
    compiler_params=_PARAMS,
    scratch_types=[
        pltpu.VMEM((NG,), jnp.float32),
        pltpu.VMEM((NG,), jnp.float32),
        pltpu.VMEM((NG,), jnp.float32),
        pltpu.VMEM((NG,), jnp.float32),
        pltpu.VMEM((NG,), jnp.float32),
        pltpu.VMEM((APW,), jnp.float32),
        pltpu.VMEM((APW,), jnp.float32),
        pltpu.VMEM((APW,), jnp.float32),
        pltpu.VMEM((APW,), jnp.float32),
        pltpu.VMEM((APW,), jnp.float32),
        pltpu.VMEM((APW,), jnp.float32),
        pltpu.VMEM((APW,), jnp.int32),
        pltpu.VMEM((16, APW), jnp.float32),
        pltpu.VMEM((NG,), jnp.float32),
        pltpu.VMEM((512,), jnp.float32),
    ],
)
def _pass_a(gx1_h, gy1_h, gx2_h, gy2_h, ax1_h, ay1_h, ax2_h, ay2_h,
            mat_h, cmax_h, cidx_h, rmax_h, tail_h,
            gx1, gy1, gx2, gy2, ga, ax1, ay1, ax2, ay2, aa,
            cmv, civ, buf, rmw, tb):
    wid = lax.axis_index("s") * NC + lax.axis_index("c")
    base = wid * APW

    pltpu.sync_copy(gx1_h, gx1)
    pltpu.sync_copy(gy1_h, gy1)
    pltpu.sync_copy(gx2_h, gx2)
    pltpu.sync_copy(gy2_h, gy2)
    pltpu.sync_copy(ax1_h.at[pl.ds(base, APW)], ax1)
    pltpu.sync_copy(ay1_h.at[pl.ds(base, APW)], ay1)
    pltpu.sync_copy(ax2_h.at[pl.ds(base, APW)], ax2)
    pltpu.sync_copy(ay2_h.at[pl.ds(base, APW)], ay2)

    def _garea(t, _):
        s = pl.ds(t * 16, 16)
        ga[s] = (gx2[s] - gx1[s]) * (gy2[s] - gy1[s])
        return 0
    lax.fori_loop(0, NT, _garea, 0)

    def _aarea(c, _):
        s = pl.ds(c * 16, 16)
        aa[s] = (ax2[s] - ax1[s]) * (ay2[s] - ay1[s])
        cmv[s] = jnp.full((16,), -1.0, jnp.float32)
        civ[s] = jnp.zeros((16,), jnp.int32)
        return 0
    lax.fori_loop(0, NCH, _aarea, 0)

    lane = jnp.arange(16, dtype=jnp.int32)
    zeros16 = jnp.zeros((16,), jnp.float32)

    def _gt_group(t, _):
        gs = pl.ds(t * 16, 16)
        g1 = gx1[gs]
        g2 = gy1[gs]
        g3 = gx2[gs]
        g4 = gy2[gs]
        gar = ga[gs]

        racc = [None] * 16
        for j in range(16):
            g1b = _bcast(g1, j)
            g2b = _bcast(g2, j)
            g3b = _bcast(g3, j)
            g4b = _bcast(g4, j)
            gab = _bcast(gar, j)
            gidx = t * 16 + j

            def _chunk(c, rj, j=j, g1b=g1b, g2b=g2b, g3b=g3b, g4b=g4b,
                       gab=gab):
                s = pl.ds(c * 16, 16)
                iou = _iou16(ax1[s], ay1[s], ax2[s], ay2[s], aa[s],
                             g1b, g2b, g3b, g4b, gab)
                buf[j, s] = iou
                return jnp.maximum(rj, iou)

            racc[j] = plsc.parallel_loop(0, NCH, unroll=2,
                                         carry=zeros16)(_chunk)

        def _cmx(c, t=t):
            s = pl.ds(c * 16, 16)
            cm = cmv[s]
            ci = civ[s]
            for j in range(16):
                v = buf[j, s]
                ci = jnp.where(v > cm, t * 16 + j, ci)
                cm = jnp.maximum(cm, v)
            cmv[s] = cm
            civ[s] = ci
        plsc.parallel_loop(0, NCH, unroll=2)(_cmx)

        @pl.when(wid != NW - 1)
        def _():
            pltpu.sync_copy(buf, mat_h.at[pl.ds(t * 16, 16), pl.ds(base, APW)])

        @pl.when(wid == NW - 1)
        def _():
            for j in range(16):
                tb[pl.ds(j * 32, 16)] = buf[j, pl.ds(128, 16)]
                tb[pl.ds(j * 32 + 16, 16)] = buf[j, pl.ds(144, 16)]
            pltpu.sync_copy(buf.at[:, pl.ds(0, 128)],
                            mat_h.at[pl.ds(t * 16, 16), pl.ds(NA - LASTW, 128)])
            pltpu.sync_copy(tb, tail_h.at[pl.ds(t * 512, 512)])

        acc = zeros16
        for j in range(16):
            m = jnp.max(racc[j])
            acc = jnp.where(lane == j, m, acc)
        rmw[gs] = acc
        return 0

    lax.fori_loop(0, NT, _gt_group, 0)

    pltpu.sync_copy(cmv, cmax_h.at[pl.ds(base, APW)])
    pltpu.sync_copy(civ, cidx_h.at[pl.ds(base, APW)])
    pltpu.sync_copy(rmw, rmax_h.at[pl.ds(wid * NG, NG)])


@functools.partial(
    pl.kernel,
    out_type=[jax.ShapeDtypeStruct((NAP,), jnp.int32)],
    mesh=_MESH,
    compiler_params=_PARAMS,
    scratch_types=[
        pltpu.VMEM((NG,), jnp.float32),
        pltpu.VMEM((NG,), jnp.float32),
        pltpu.VMEM((NG,), jnp.float32),
        pltpu.VMEM((NG,), jnp.float32),
        pltpu.VMEM((NG,), jnp.float32),
        pltpu.VMEM((APW,), jnp.float32),
        pltpu.VMEM((APW,), jnp.float32),
        pltpu.VMEM((APW,), jnp.float32),
        pltpu.VMEM((APW,), jnp.float32),
        pltpu.VMEM((APW,), jnp.float32),
        pltpu.VMEM((NW * NG,), jnp.float32),
        pltpu.VMEM((NG,), jnp.float32),
        pltpu.VMEM((APW,), jnp.float32),
        pltpu.VMEM((APW,), jnp.int32),
        pltpu.VMEM((APW,), jnp.int32),
        pltpu.VMEM((APW,), jnp.int32),
    ],
)
def _pass_b(gx1_h, gy1_h, gx2_h, gy2_h, ax1_h, ay1_h, ax2_h, ay2_h,
            rmax_h, cmax_h, cidx_h, idx_h,
            gx1, gy1, gx2, gy2, ga, ax1, ay1, ax2, ay2, aa,
            rall, grm, cmv, civ, resv, outv):
    wid = lax.axis_index("s") * NC + lax.axis_index("c")
    base = wid * APW

    pltpu.sync_copy(gx1_h, gx1)
    pltpu.sync_copy(gy1_h, gy1)
    pltpu.sync_copy(gx2_h, gx2)
    pltpu.sync_copy(gy2_h, gy2)
    pltpu.sync_copy(ax1_h.at[pl.ds(base, APW)], ax1)
    pltpu.sync_copy(ay1_h.at[pl.ds(base, APW)], ay1)
    pltpu.sync_copy(ax2_h.at[pl.ds(base, APW)], ax2)
    pltpu.sync_copy(ay2_h.at[pl.ds(base, APW)], ay2)
    pltpu.sync_copy(rmax_h, rall)
    pltpu.sync_copy(cmax_h.at[pl.ds(base, APW)], cmv)
    pltpu.sync_copy(cidx_h.at[pl.ds(base, APW)], civ)

    def _garea(t, _):
        s = pl.ds(t * 16, 16)
        ga[s] = (gx2[s] - gx1[s]) * (gy2[s] - gy1[s])
        return 0
    lax.fori_loop(0, NT, _garea, 0)

    def _aareab(c, _):
        s = pl.ds(c * 16, 16)
        aa[s] = (ax2[s] - ax1[s]) * (ay2[s] - ay1[s])
        resv[s] = jnp.zeros((16,), jnp.int32)
        return 0
    lax.fori_loop(0, NCH, _aareab, 0)

    def _reduce(t, _):
        acc = rall[pl.ds(t * 16, 16)]
        for w in range(1, NW):
            acc = jnp.maximum(acc, rall[pl.ds(w * NG + t * 16, 16)])
        grm[pl.ds(t * 16, 16)] = acc
        return 0
    lax.fori_loop(0, NT, _reduce, 0)

    lane = jnp.arange(16, dtype=jnp.int32)
    ones16 = jnp.ones((16,), jnp.int32)

    def _mark(t, _):
        gs = pl.ds(t * 16, 16)
        grmv = grm[gs]
        ownv = rall[pl.ds(wid * NG + t * 16, 16)]
        eq = ownv == grmv
        g1 = gx1[gs]
        g2 = gy1[gs]
        g3 = gx2[gs]
        g4 = gy2[gs]
        gar = ga[gs]
        for j in range(16):
            cand = jnp.any(eq & (lane == j))

            @pl.when(cand)
            def _(j=j, g1=g1, g2=g2, g3=g3, g4=g4, gar=gar, grmv=grmv):
                g1b = _bcast(g1, j)
                g2b = _bcast(g2, j)
                g3b = _bcast(g3, j)
                g4b = _bcast(g4, j)
                gab = _bcast(gar, j)
                gmb = _bcast(grmv, j)

                def _scan(c, _):
                    s = pl.ds(c * 16, 16)
                    iou = _iou16(ax1[s], ay1[s], ax2[s], ay2[s], aa[s],
                                 g1b, g2b, g3b, g4b, gab)
                    plsc.store_scatter(resv, [lane + c * 16], ones16,
                                       mask=iou == gmb)
                    return 0
                lax.fori_loop(0, NCH, _scan, 0)
        return 0

    lax.fori_loop(0, NT, _mark, 0)

    def _final(c, _):
        s = pl.ds(c * 16, 16)
        mx = cmv[s]
        ix = civ[s]
        m = jnp.where(mx < BG, jnp.int32(-1), ix)
        m = jnp.where((mx >= BG) & (mx < FG), jnp.int32(-2), m)
        m = jnp.where(resv[s] > 0, ix, m)
        outv[s] = m
        return 0
    lax.fori_loop(0, NCH, _final, 0)

    pltpu.sync_copy(outv, idx_h.at[pl.ds(base, APW)])


def kernel(gt_boxes, anchors):
    ap = jnp.concatenate(
        [anchors, jnp.zeros((NAP - NA, 4), jnp.float32)], axis=0)
    gcols = (gt_boxes[:, 0], gt_boxes[:, 1], gt_boxes[:, 2], gt_boxes[:, 3])
    acols = (ap[:, 0], ap[:, 1], ap[:, 2], ap[:, 3])
    mat, cmax, cidx, rmax, tail = _pass_a(*gcols, *acols)
    idxp, = _pass_b(*gcols, *acols, rmax, cmax, cidx)
    mat = lax.dynamic_update_slice(mat, tail.reshape(NG, 32), (0, NA - 32))
    return mat, idxp[:NA].astype(jnp.int64)


# speedup vs baseline: 1.5869x; 1.1531x over previous
import functools

import jax
import jax.numpy as jnp
from jax import lax
from jax.experimental import pallas as pl
from jax.experimental.pallas import tpu as pltpu
from jax.experimental.pallas import tpu_sc as plsc

NG = 512
NA = 20000
NC = 2
NS = 16
NW = NC * NS
APW = 640
NAP = NW * APW
LASTW = NA - (NW - 1) * APW
NT = NG // 16
NCH = APW // 16
BG = 0.4
FG = 0.5

_MESH = plsc.VectorSubcoreMesh(core_axis_name="c", subcore_axis_name="s")
_PARAMS = pltpu.CompilerParams(needs_layout_passes=False)


def _bcast(v, j):
    idx = jnp.full((16,), j, jnp.int32)
    return v.at[idx].get(mode="promise_in_bounds")


def _iou16(a1, a2, a3, a4, aar, g1b, g2b, g3b, g4b, gab):
    ltx = jnp.maximum(a1, g1b)
    lty = jnp.maximum(a2, g2b)
    rbx = jnp.minimum(a3, g3b)
    rby = jnp.minimum(a4, g4b)
    w = jnp.maximum(rbx - ltx, 0.0)
    h = jnp.maximum(rby - lty, 0.0)
    inter = w * h
    union = (gab + aar) - inter
    return inter / union


@functools.partial(
    pl.kernel,
    out_type=[
        jax.ShapeDtypeStruct((NG, NA), jnp.float32),
        jax.ShapeDtypeStruct((NAP,), jnp.float32),
        jax.ShapeDtypeStruct((NAP,), jnp.int32),
        jax.ShapeDtypeStruct((NW * NG,), jnp.float32),
        jax.ShapeDtypeStruct((NG * 32,), jnp.float32),
    ],
    mesh=_MESH,
    compiler_params=_PARAMS,
    scratch_types=[
        pltpu.VMEM((NG,), jnp.float32),
        pltpu.VMEM((NG,), jnp.float32),
        pltpu.VMEM((NG,), jnp.float32),
        pltpu.VMEM((NG,), jnp.float32),
        pltpu.VMEM((NG,), jnp.float32),
        pltpu.VMEM((APW,), jnp.float32),
        pltpu.VMEM((APW,), jnp.float32),
        pltpu.VMEM((APW,), jnp.float32),
        pltpu.VMEM((APW,), jnp.float32),
        pltpu.VMEM((APW,), jnp.float32),
        pltpu.VMEM((APW,), jnp.int32),
        pltpu.VMEM((16, APW), jnp.float32),
        pltpu.VMEM((NG,), jnp.float32),
        pltpu.VMEM((512,), jnp.float32),
        pltpu.SemaphoreType.DMA,
    ],
)
def _pass_a(gx1_h, gy1_h, gx2_h, gy2_h, ax1_h, ay1_h, ax2_h, ay2_h,
            mat_h, cmax_h, cidx_h, rmax_h, tail_h,
            gx1, gy1, gx2, gy2, ga, ax1, ay1, ax2, ay2,
            cmv, civ, buf, rmw, tb, dsem):
    wid = lax.axis_index("s") * NC + lax.axis_index("c")
    base = wid * APW

    cps = [pltpu.async_copy(gx1_h, gx1, dsem),
           pltpu.async_copy(gy1_h, gy1, dsem),
           pltpu.async_copy(gx2_h, gx2, dsem),
           pltpu.async_copy(gy2_h, gy2, dsem),
           pltpu.async_copy(ax1_h.at[pl.ds(base, APW)], ax1, dsem),
           pltpu.async_copy(ay1_h.at[pl.ds(base, APW)], ay1, dsem),
           pltpu.async_copy(ax2_h.at[pl.ds(base, APW)], ax2, dsem),
           pltpu.async_copy(ay2_h.at[pl.ds(base, APW)], ay2, dsem)]
    for cp in cps:
        cp.wait()

    def _garea(t, _):
        s = pl.ds(t * 16, 16)
        ga[s] = (gx2[s] - gx1[s]) * (gy2[s] - gy1[s])
        return 0
    lax.fori_loop(0, NT, _garea, 0)

    def _cminit(c, _):
        s = pl.ds(c * 16, 16)
        cmv[s] = jnp.full((16,), -1.0, jnp.float32)
        civ[s] = jnp.zeros((16,), jnp.int32)
        return 0
    lax.fori_loop(0, NCH, _cminit, 0)

    lane = jnp.arange(16, dtype=jnp.int32)
    zeros16 = jnp.zeros((16,), jnp.float32)

    def _gt_group(t, _):
        gs = pl.ds(t * 16, 16)
        g1 = gx1[gs]
        g2 = gy1[gs]
        g3 = gx2[gs]
        g4 = gy2[gs]
        gar = ga[gs]

        racc = [None] * 16
        for j in range(16):
            g1b = _bcast(g1, j)
            g2b = _bcast(g2, j)
            g3b = _bcast(g3, j)
            g4b = _bcast(g4, j)
            gab = _bcast(gar, j)
            gidx = t * 16 + j

            def _chunk(c, rj, j=j, g1b=g1b, g2b=g2b, g3b=g3b, g4b=g4b,
                       gab=gab):
                s = pl.ds(c * 16, 16)
                a1 = ax1[s]
                a2 = ay1[s]
                a3 = ax2[s]
                a4 = ay2[s]
                aar = (a3 - a1) * (a4 - a2)
                iou = _iou16(a1, a2, a3, a4, aar,
                             g1b, g2b, g3b, g4b, gab)
                buf[j, s] = iou
                return jnp.maximum(rj, iou)

            racc[j] = plsc.parallel_loop(0, NCH, unroll=2,
                                         carry=zeros16)(_chunk)

        def _cmx(c, t=t):
            s = pl.ds(c * 16, 16)
            cm = cmv[s]
            ci = civ[s]
            for j in range(16):
                v = buf[j, s]
                ci = jnp.where(v > cm, t * 16 + j, ci)
                cm = jnp.maximum(cm, v)
            cmv[s] = cm
            civ[s] = ci
        plsc.parallel_loop(0, NCH, unroll=2)(_cmx)

        @pl.when(wid != NW - 1)
        def _():
            pltpu.sync_copy(buf, mat_h.at[pl.ds(t * 16, 16), pl.ds(base, APW)])

        @pl.when(wid == NW - 1)
        def _():
            for j in range(16):
                tb[pl.ds(j * 32, 16)] = buf[j, pl.ds(128, 16)]
                tb[pl.ds(j * 32 + 16, 16)] = buf[j, pl.ds(144, 16)]
            pltpu.sync_copy(buf.at[:, pl.ds(0, 128)],
                            mat_h.at[pl.ds(t * 16, 16), pl.ds(NA - LASTW, 128)])
            pltpu.sync_copy(tb, tail_h.at[pl.ds(t * 512, 512)])

        acc = zeros16
        for j in range(16):
            m = jnp.max(racc[j])
            acc = jnp.where(lane == j, m, acc)
        rmw[gs] = acc
        return 0

    lax.fori_loop(0, NT, _gt_group, 0)

    pltpu.sync_copy(cmv, cmax_h.at[pl.ds(base, APW)])
    pltpu.sync_copy(civ, cidx_h.at[pl.ds(base, APW)])
    pltpu.sync_copy(rmw, rmax_h.at[pl.ds(wid * NG, NG)])


@functools.partial(
    pl.kernel,
    out_type=[jax.ShapeDtypeStruct((NAP,), jnp.int32)],
    mesh=_MESH,
    compiler_params=_PARAMS,
    scratch_types=[
        pltpu.VMEM((NG,), jnp.float32),
        pltpu.VMEM((NG,), jnp.float32),
        pltpu.VMEM((NG,), jnp.float32),
        pltpu.VMEM((NG,), jnp.float32),
        pltpu.VMEM((NG,), jnp.float32),
        pltpu.VMEM((APW,), jnp.float32),
        pltpu.VMEM((APW,), jnp.float32),
        pltpu.VMEM((APW,), jnp.float32),
        pltpu.VMEM((APW,), jnp.float32),
        pltpu.VMEM((APW,), jnp.float32),
        pltpu.VMEM((NW * NG,), jnp.float32),
        pltpu.VMEM((NG,), jnp.float32),
        pltpu.VMEM((APW,), jnp.float32),
        pltpu.VMEM((APW,), jnp.int32),
        pltpu.VMEM((APW,), jnp.int32),
        pltpu.VMEM((APW,), jnp.int32),
        pltpu.SemaphoreType.DMA,
    ],
)
def _pass_b(gx1_h, gy1_h, gx2_h, gy2_h, ax1_h, ay1_h, ax2_h, ay2_h,
            rmax_h, cmax_h, cidx_h, idx_h,
            gx1, gy1, gx2, gy2, ga, ax1, ay1, ax2, ay2, aa,
            rall, grm, cmv, civ, resv, outv, dsem):
    wid = lax.axis_index("s") * NC + lax.axis_index("c")
    base = wid * APW

    cps = [pltpu.async_copy(gx1_h, gx1, dsem),
           pltpu.async_copy(gy1_h, gy1, dsem),
           pltpu.async_copy(gx2_h, gx2, dsem),
           pltpu.async_copy(gy2_h, gy2, dsem),
           pltpu.async_copy(ax1_h.at[pl.ds(base, APW)], ax1, dsem),
           pltpu.async_copy(ay1_h.at[pl.ds(base, APW)], ay1, dsem),
           pltpu.async_copy(ax2_h.at[pl.ds(base, APW)], ax2, dsem),
           pltpu.async_copy(ay2_h.at[pl.ds(base, APW)], ay2, dsem),
           pltpu.async_copy(rmax_h, rall, dsem),
           pltpu.async_copy(cmax_h.at[pl.ds(base, APW)], cmv, dsem),
           pltpu.async_copy(cidx_h.at[pl.ds(base, APW)], civ, dsem)]
    for cp in cps:
        cp.wait()

    def _garea(t, _):
        s = pl.ds(t * 16, 16)
        ga[s] = (gx2[s] - gx1[s]) * (gy2[s] - gy1[s])
        return 0
    lax.fori_loop(0, NT, _garea, 0)

    def _aareab(c, _):
        s = pl.ds(c * 16, 16)
        aa[s] = (ax2[s] - ax1[s]) * (ay2[s] - ay1[s])
        resv[s] = jnp.zeros((16,), jnp.int32)
        return 0
    lax.fori_loop(0, NCH, _aareab, 0)

    def _reduce(t, _):
        acc = rall[pl.ds(t * 16, 16)]
        for w in range(1, NW):
            acc = jnp.maximum(acc, rall[pl.ds(w * NG + t * 16, 16)])
        grm[pl.ds(t * 16, 16)] = acc
        return 0
    lax.fori_loop(0, NT, _reduce, 0)

    lane = jnp.arange(16, dtype=jnp.int32)
    ones16 = jnp.ones((16,), jnp.int32)

    def _mark(t, _):
        gs = pl.ds(t * 16, 16)
        grmv = grm[gs]
        ownv = rall[pl.ds(wid * NG + t * 16, 16)]
        eq = ownv == grmv
        g1 = gx1[gs]
        g2 = gy1[gs]
        g3 = gx2[gs]
        g4 = gy2[gs]
        gar = ga[gs]
        for j in range(16):
            cand = jnp.any(eq & (lane == j))

            @pl.when(cand)
            def _(j=j, g1=g1, g2=g2, g3=g3, g4=g4, gar=gar, grmv=grmv):
                g1b = _bcast(g1, j)
                g2b = _bcast(g2, j)
                g3b = _bcast(g3, j)
                g4b = _bcast(g4, j)
                gab = _bcast(gar, j)
                gmb = _bcast(grmv, j)

                def _scan(c, _):
                    s = pl.ds(c * 16, 16)
                    iou = _iou16(ax1[s], ay1[s], ax2[s], ay2[s], aa[s],
                                 g1b, g2b, g3b, g4b, gab)
                    plsc.store_scatter(resv, [lane + c * 16], ones16,
                                       mask=iou == gmb)
                    return 0
                lax.fori_loop(0, NCH, _scan, 0)
        return 0

    lax.fori_loop(0, NT, _mark, 0)

    def _final(c, _):
        s = pl.ds(c * 16, 16)
        mx = cmv[s]
        ix = civ[s]
        m = jnp.where(mx < BG, jnp.int32(-1), ix)
        m = jnp.where((mx >= BG) & (mx < FG), jnp.int32(-2), m)
        m = jnp.where(resv[s] > 0, ix, m)
        outv[s] = m
        return 0
    lax.fori_loop(0, NCH, _final, 0)

    pltpu.sync_copy(outv, idx_h.at[pl.ds(base, APW)])


def _tail_body(tail_ref, mat_ref, out_ref):
    out_ref[:, 0:32] = tail_ref[:, :]


_tail_merge = pl.pallas_call(
    _tail_body,
    out_shape=jax.ShapeDtypeStruct((NG, NA), jnp.float32),
    grid=(1,),
    in_specs=[
        pl.BlockSpec((NG, 32), lambda i: (0, 0)),
        pl.BlockSpec((NG, 128), lambda i: (0, (NA - 32) // 128)),
    ],
    out_specs=pl.BlockSpec((NG, 128), lambda i: (0, (NA - 32) // 128)),
    input_output_aliases={1: 0},
)


def kernel(gt_boxes, anchors):
    ap = jnp.concatenate(
        [anchors, jnp.zeros((NAP - NA, 4), jnp.float32)], axis=0)
    gcols = (gt_boxes[:, 0], gt_boxes[:, 1], gt_boxes[:, 2], gt_boxes[:, 3])
    acols = (ap[:, 0], ap[:, 1], ap[:, 2], ap[:, 3])
    mat, cmax, cidx, rmax, tail = _pass_a(*gcols, *acols)
    idxp, = _pass_b(*gcols, *acols, rmax, cmax, cidx)
    mat = _tail_merge(tail.reshape(NG, 32), mat)
    return mat, idxp[:NA].astype(jnp.int64)


# speedup vs baseline: 1.6378x; 1.0320x over previous
import functools

import jax
import jax.numpy as jnp
from jax import lax
from jax.experimental import pallas as pl
from jax.experimental.pallas import tpu as pltpu
from jax.experimental.pallas import tpu_sc as plsc

NG = 512
NA = 20000
NC = 2
NS = 16
NW = NC * NS
APW = 640
NAP = NW * APW
LASTW = NA - (NW - 1) * APW
NT = NG // 16
NCH = APW // 16
BG = 0.4
FG = 0.5

_MESH = plsc.VectorSubcoreMesh(core_axis_name="c", subcore_axis_name="s")
_PARAMS = pltpu.CompilerParams(needs_layout_passes=False)


def _bcast(v, j):
    idx = jnp.full((16,), j, jnp.int32)
    return v.at[idx].get(mode="promise_in_bounds")


def _iou16(a1, a2, a3, a4, aar, g1b, g2b, g3b, g4b, gab):
    ltx = jnp.maximum(a1, g1b)
    lty = jnp.maximum(a2, g2b)
    rbx = jnp.minimum(a3, g3b)
    rby = jnp.minimum(a4, g4b)
    w = jnp.maximum(rbx - ltx, 0.0)
    h = jnp.maximum(rby - lty, 0.0)
    inter = w * h
    union = (gab + aar) - inter
    return inter / union


@functools.partial(
    pl.kernel,
    out_type=[
        jax.ShapeDtypeStruct((NG, NA), jnp.float32),
        jax.ShapeDtypeStruct((NAP,), jnp.float32),
        jax.ShapeDtypeStruct((NAP,), jnp.int32),
        jax.ShapeDtypeStruct((NW * NG,), jnp.float32),
        jax.ShapeDtypeStruct((NG * 32,), jnp.float32),
    ],
    mesh=_MESH,
    compiler_params=_PARAMS,
    scratch_types=[
        pltpu.VMEM((NG,), jnp.float32),
        pltpu.VMEM((NG,), jnp.float32),
        pltpu.VMEM((NG,), jnp.float32),
        pltpu.VMEM((NG,), jnp.float32),
        pltpu.VMEM((NG,), jnp.float32),
        pltpu.VMEM((APW,), jnp.float32),
        pltpu.VMEM((APW,), jnp.float32),
        pltpu.VMEM((APW,), jnp.float32),
        pltpu.VMEM((APW,), jnp.float32),
        pltpu.VMEM((APW,), jnp.float32),
        pltpu.VMEM((APW,), jnp.int32),
        pltpu.VMEM((16, APW), jnp.float32),
        pltpu.VMEM((16, APW), jnp.float32),
        pltpu.VMEM((NG,), jnp.float32),
        pltpu.VMEM((512,), jnp.float32),
        pltpu.VMEM((512,), jnp.float32),
        pltpu.SemaphoreType.DMA,
        pltpu.SemaphoreType.DMA,
        pltpu.SemaphoreType.DMA,
    ],
)
def _pass_a(gx1_h, gy1_h, gx2_h, gy2_h, ax1_h, ay1_h, ax2_h, ay2_h,
            mat_h, cmax_h, cidx_h, rmax_h, tail_h,
            gx1, gy1, gx2, gy2, ga, ax1, ay1, ax2, ay2,
            cmv, civ, buf0, buf1, rmw, tb0, tb1, dsem, osem0, osem1):
    wid = lax.axis_index("s") * NC + lax.axis_index("c")
    base = wid * APW

    cps = [pltpu.async_copy(gx1_h, gx1, dsem),
           pltpu.async_copy(gy1_h, gy1, dsem),
           pltpu.async_copy(gx2_h, gx2, dsem),
           pltpu.async_copy(gy2_h, gy2, dsem),
           pltpu.async_copy(ax1_h.at[pl.ds(base, APW)], ax1, dsem),
           pltpu.async_copy(ay1_h.at[pl.ds(base, APW)], ay1, dsem),
           pltpu.async_copy(ax2_h.at[pl.ds(base, APW)], ax2, dsem),
           pltpu.async_copy(ay2_h.at[pl.ds(base, APW)], ay2, dsem)]
    for cp in cps:
        cp.wait()

    def _garea(t, _):
        s = pl.ds(t * 16, 16)
        ga[s] = (gx2[s] - gx1[s]) * (gy2[s] - gy1[s])
        return 0
    lax.fori_loop(0, NT, _garea, 0)

    def _cminit(c, _):
        s = pl.ds(c * 16, 16)
        cmv[s] = jnp.full((16,), -1.0, jnp.float32)
        civ[s] = jnp.zeros((16,), jnp.int32)
        return 0
    lax.fori_loop(0, NCH, _cminit, 0)

    lane = jnp.arange(16, dtype=jnp.int32)
    zeros16 = jnp.zeros((16,), jnp.float32)

    def _do_group(t, buf, tb, osem):
        gs = pl.ds(t * 16, 16)
        g1 = gx1[gs]
        g2 = gy1[gs]
        g3 = gx2[gs]
        g4 = gy2[gs]
        gar = ga[gs]

        racc = [None] * 16
        for j in range(16):
            g1b = _bcast(g1, j)
            g2b = _bcast(g2, j)
            g3b = _bcast(g3, j)
            g4b = _bcast(g4, j)
            gab = _bcast(gar, j)

            def _chunk(c, rj, j=j, g1b=g1b, g2b=g2b, g3b=g3b, g4b=g4b,
                       gab=gab, buf=buf):
                s = pl.ds(c * 16, 16)
                a1 = ax1[s]
                a2 = ay1[s]
                a3 = ax2[s]
                a4 = ay2[s]
                aar = (a3 - a1) * (a4 - a2)
                iou = _iou16(a1, a2, a3, a4, aar,
                             g1b, g2b, g3b, g4b, gab)
                buf[j, s] = iou
                return jnp.maximum(rj, iou)

            racc[j] = plsc.parallel_loop(0, NCH, unroll=2,
                                         carry=zeros16)(_chunk)

        def _cmx(c, t=t, buf=buf):
            s = pl.ds(c * 16, 16)
            cm = cmv[s]
            ci = civ[s]
            for j in range(16):
                v = buf[j, s]
                ci = jnp.where(v > cm, t * 16 + j, ci)
                cm = jnp.maximum(cm, v)
            cmv[s] = cm
            civ[s] = ci
        plsc.parallel_loop(0, NCH, unroll=2)(_cmx)

        @pl.when(wid != NW - 1)
        def _():
            pltpu.async_copy(
                buf, mat_h.at[pl.ds(t * 16, 16), pl.ds(base, APW)], osem)

        @pl.when(wid == NW - 1)
        def _():
            for j in range(16):
                tb[pl.ds(j * 32, 16)] = buf[j, pl.ds(128, 16)]
                tb[pl.ds(j * 32 + 16, 16)] = buf[j, pl.ds(144, 16)]
            pltpu.async_copy(
                buf.at[:, pl.ds(0, 128)],
                mat_h.at[pl.ds(t * 16, 16), pl.ds(NA - LASTW, 128)], osem)
            pltpu.async_copy(tb, tail_h.at[pl.ds(t * 512, 512)], osem)

        acc = zeros16
        for j in range(16):
            m = jnp.max(racc[j])
            acc = jnp.where(lane == j, m, acc)
        rmw[gs] = acc

    def _drain(buf, tb, osem):
        @pl.when(wid != NW - 1)
        def _():
            pltpu.make_async_copy(
                buf, mat_h.at[pl.ds(0, 16), pl.ds(base, APW)], osem).wait()

        @pl.when(wid == NW - 1)
        def _():
            pltpu.make_async_copy(
                buf.at[:, pl.ds(0, 128)],
                mat_h.at[pl.ds(0, 16), pl.ds(NA - LASTW, 128)], osem).wait()
            pltpu.make_async_copy(tb, tail_h.at[pl.ds(0, 512)], osem).wait()

    def _gt_pair(tt, _):
        @pl.when(tt > 0)
        def _():
            _drain(buf0, tb0, osem0)
        _do_group(tt * 2, buf0, tb0, osem0)

        @pl.when(tt > 0)
        def _():
            _drain(buf1, tb1, osem1)
        _do_group(tt * 2 + 1, buf1, tb1, osem1)
        return 0

    lax.fori_loop(0, NT // 2, _gt_pair, 0)
    _drain(buf0, tb0, osem0)
    _drain(buf1, tb1, osem1)

    pltpu.sync_copy(cmv, cmax_h.at[pl.ds(base, APW)])
    pltpu.sync_copy(civ, cidx_h.at[pl.ds(base, APW)])
    pltpu.sync_copy(rmw, rmax_h.at[pl.ds(wid * NG, NG)])


@functools.partial(
    pl.kernel,
    out_type=[jax.ShapeDtypeStruct((NAP,), jnp.int32)],
    mesh=_MESH,
    compiler_params=_PARAMS,
    scratch_types=[
        pltpu.VMEM((NG,), jnp.float32),
        pltpu.VMEM((NG,), jnp.float32),
        pltpu.VMEM((NG,), jnp.float32),
        pltpu.VMEM((NG,), jnp.float32),
        pltpu.VMEM((NG,), jnp.float32),
        pltpu.VMEM((APW,), jnp.float32),
        pltpu.VMEM((APW,), jnp.float32),
        pltpu.VMEM((APW,), jnp.float32),
        pltpu.VMEM((APW,), jnp.float32),
        pltpu.VMEM((APW,), jnp.float32),
        pltpu.VMEM((NW * NG,), jnp.float32),
        pltpu.VMEM((NG,), jnp.float32),
        pltpu.VMEM((APW,), jnp.float32),
        pltpu.VMEM((APW,), jnp.int32),
        pltpu.VMEM((APW,), jnp.int32),
        pltpu.VMEM((APW,), jnp.int32),
        pltpu.SemaphoreType.DMA,
    ],
)
def _pass_b(gx1_h, gy1_h, gx2_h, gy2_h, ax1_h, ay1_h, ax2_h, ay2_h,
            rmax_h, cmax_h, cidx_h, idx_h,
            gx1, gy1, gx2, gy2, ga, ax1, ay1, ax2, ay2, aa,
            rall, grm, cmv, civ, resv, outv, dsem):
    wid = lax.axis_index("s") * NC + lax.axis_index("c")
    base = wid * APW

    cps = [pltpu.async_copy(gx1_h, gx1, dsem),
           pltpu.async_copy(gy1_h, gy1, dsem),
           pltpu.async_copy(gx2_h, gx2, dsem),
           pltpu.async_copy(gy2_h, gy2, dsem),
           pltpu.async_copy(ax1_h.at[pl.ds(base, APW)], ax1, dsem),
           pltpu.async_copy(ay1_h.at[pl.ds(base, APW)], ay1, dsem),
           pltpu.async_copy(ax2_h.at[pl.ds(base, APW)], ax2, dsem),
           pltpu.async_copy(ay2_h.at[pl.ds(base, APW)], ay2, dsem),
           pltpu.async_copy(rmax_h, rall, dsem),
           pltpu.async_copy(cmax_h.at[pl.ds(base, APW)], cmv, dsem),
           pltpu.async_copy(cidx_h.at[pl.ds(base, APW)], civ, dsem)]
    for cp in cps:
        cp.wait()

    def _garea(t, _):
        s = pl.ds(t * 16, 16)
        ga[s] = (gx2[s] - gx1[s]) * (gy2[s] - gy1[s])
        return 0
    lax.fori_loop(0, NT, _garea, 0)

    def _aareab(c, _):
        s = pl.ds(c * 16, 16)
        aa[s] = (ax2[s] - ax1[s]) * (ay2[s] - ay1[s])
        resv[s] = jnp.zeros((16,), jnp.int32)
        return 0
    lax.fori_loop(0, NCH, _aareab, 0)

    def _reduce(t, _):
        acc = rall[pl.ds(t * 16, 16)]
        for w in range(1, NW):
            acc = jnp.maximum(acc, rall[pl.ds(w * NG + t * 16, 16)])
        grm[pl.ds(t * 16, 16)] = acc
        return 0
    lax.fori_loop(0, NT, _reduce, 0)

    lane = jnp.arange(16, dtype=jnp.int32)
    ones16 = jnp.ones((16,), jnp.int32)

    def _mark(t, _):
        gs = pl.ds(t * 16, 16)
        grmv = grm[gs]
        ownv = rall[pl.ds(wid * NG + t * 16, 16)]
        eq = ownv == grmv
        g1 = gx1[gs]
        g2 = gy1[gs]
        g3 = gx2[gs]
        g4 = gy2[gs]
        gar = ga[gs]
        for j in range(16):
            cand = jnp.any(eq & (lane == j))

            @pl.when(cand)
            def _(j=j, g1=g1, g2=g2, g3=g3, g4=g4, gar=gar, grmv=grmv):
                g1b = _bcast(g1, j)
                g2b = _bcast(g2, j)
                g3b = _bcast(g3, j)
                g4b = _bcast(g4, j)
                gab = _bcast(gar, j)
                gmb = _bcast(grmv, j)

                def _scan(c, _):
                    s = pl.ds(c * 16, 16)
                    iou = _iou16(ax1[s], ay1[s], ax2[s], ay2[s], aa[s],
                                 g1b, g2b, g3b, g4b, gab)
                    plsc.store_scatter(resv, [lane + c * 16], ones16,
                                       mask=iou == gmb)
                    return 0
                lax.fori_loop(0, NCH, _scan, 0)
        return 0

    lax.fori_loop(0, NT, _mark, 0)

    def _final(c, _):
        s = pl.ds(c * 16, 16)
        mx = cmv[s]
        ix = civ[s]
        m = jnp.where(mx < BG, jnp.int32(-1), ix)
        m = jnp.where((mx >= BG) & (mx < FG), jnp.int32(-2), m)
        m = jnp.where(resv[s] > 0, ix, m)
        outv[s] = m
        return 0
    lax.fori_loop(0, NCH, _final, 0)

    pltpu.sync_copy(outv, idx_h.at[pl.ds(base, APW)])


def _tail_body(tail_ref, mat_ref, out_ref):
    out_ref[:, 0:32] = tail_ref[:, :]


_tail_merge = pl.pallas_call(
    _tail_body,
    out_shape=jax.ShapeDtypeStruct((NG, NA), jnp.float32),
    grid=(1,),
    in_specs=[
        pl.BlockSpec((NG, 32), lambda i: (0, 0)),
        pl.BlockSpec((NG, 128), lambda i: (0, (NA - 32) // 128)),
    ],
    out_specs=pl.BlockSpec((NG, 128), lambda i: (0, (NA - 32) // 128)),
    input_output_aliases={1: 0},
)


def kernel(gt_boxes, anchors):
    ap = jnp.concatenate(
        [anchors, jnp.zeros((NAP - NA, 4), jnp.float32)], axis=0)
    gcols = (gt_boxes[:, 0], gt_boxes[:, 1], gt_boxes[:, 2], gt_boxes[:, 3])
    acols = (ap[:, 0], ap[:, 1], ap[:, 2], ap[:, 3])
    mat, cmax, cidx, rmax, tail = _pass_a(*gcols, *acols)
    idxp, = _pass_b(*gcols, *acols, rmax, cmax, cidx)
    mat = _tail_merge(tail.reshape(NG, 32), mat)
    return mat, idxp[:NA].astype(jnp.int64)
